# baseline (speedup 1.0000x reference)
import functools

import jax
import jax.numpy as jnp
from jax import lax
from jax.experimental import pallas as pl
from jax.experimental.pallas import tpu as pltpu
from jax.experimental.pallas import tpu_sc as plsc

NEG_SLOPE = 0.2
EPS = 1e-16
NC = 2
NS = 16
NW = NC * NS
L = 16
G = 64
D = 128
DP = D // 2
NPAD = 240


def _mm2_body_f32(x_ref, wa_ref, wb_ref, oa_ref, ob_ref):
    xv = x_ref[...]
    oa_ref[...] = jnp.dot(xv, wa_ref[...], preferred_element_type=jnp.float32)
    ob_ref[...] = jnp.dot(xv, wb_ref[...], preferred_element_type=jnp.float32)


def _mm2_body_bf16(x_ref, wa_ref, wb_ref, oa_ref, ob_ref):
    xv = x_ref[...]
    oa_ref[...] = jnp.dot(
        xv, wa_ref[...], preferred_element_type=jnp.float32
    ).astype(jnp.bfloat16)
    ob_ref[...] = jnp.dot(
        xv, wb_ref[...], preferred_element_type=jnp.float32
    ).astype(jnp.bfloat16)


def _mm2(x, wa, wb, block, out_dtype):
    m, k = x.shape
    n = wa.shape[1]
    body = _mm2_body_bf16 if out_dtype == jnp.bfloat16 else _mm2_body_f32
    return pl.pallas_call(
        body,
        grid=(m // block,),
        in_specs=[pl.BlockSpec((block, k), lambda i: (i, 0)),
                  pl.BlockSpec((k, n), lambda i: (0, 0)),
                  pl.BlockSpec((k, n), lambda i: (0, 0))],
        out_specs=[pl.BlockSpec((block, n), lambda i: (i, 0)),
                   pl.BlockSpec((block, n), lambda i: (i, 0))],
        out_shape=[jax.ShapeDtypeStruct((m, n), out_dtype),
                   jax.ShapeDtypeStruct((m, n), out_dtype)],
    )(x, wa, wb)


def _combine_body(op_ref, dp_ref, b_ref, wa_ref, wb_ref, oa_ref, ob_ref):
    o = op_ref[0] + op_ref[1]
    den = dp_ref[0] + dp_ref[1] + EPS
    h = jax.nn.relu(o / den + b_ref[...])
    oa_ref[...] = jnp.dot(h, wa_ref[...], preferred_element_type=jnp.float32)
    ob_ref[...] = jnp.dot(h, wb_ref[...], preferred_element_type=jnp.float32)


def _combine_mm2(op, dp, b, wa, wb, block):
    n_nodes = op.shape[1]
    return pl.pallas_call(
        _combine_body,
        grid=(n_nodes // block,),
        in_specs=[pl.BlockSpec((2, block, D), lambda i: (0, i, 0)),
                  pl.BlockSpec((2, block, 1), lambda i: (0, i, 0)),
                  pl.BlockSpec((1, D), lambda i: (0, 0)),
                  pl.BlockSpec((D, D), lambda i: (0, 0)),
                  pl.BlockSpec((D, D), lambda i: (0, 0))],
        out_specs=[pl.BlockSpec((block, D), lambda i: (i, 0)),
                   pl.BlockSpec((block, D), lambda i: (i, 0))],
        out_shape=[jax.ShapeDtypeStruct((n_nodes, D), jnp.float32),
                   jax.ShapeDtypeStruct((n_nodes, D), jnp.float32)],
    )(op, dp, b.reshape(1, D), wa, wb)


def _final_body(op_ref, dp_ref, b_ref, wh_ref, bh_ref, o_ref):
    o = op_ref[0] + op_ref[1]
    den = dp_ref[0] + dp_ref[1] + EPS
    h = jax.nn.relu(o / den + b_ref[...])
    o_ref[...] = (jnp.dot(h, wh_ref[...], preferred_element_type=jnp.float32)
                  + bh_ref[...])


def _final(op, dp, b, wh, bh, block):
    n_nodes = op.shape[1]
    d_out = wh.shape[1]
    return pl.pallas_call(
        _final_body,
        grid=(n_nodes // block,),
        in_specs=[pl.BlockSpec((2, block, D), lambda i: (0, i, 0)),
                  pl.BlockSpec((2, block, 1), lambda i: (0, i, 0)),
                  pl.BlockSpec((1, D), lambda i: (0, 0)),
                  pl.BlockSpec((D, d_out), lambda i: (0, 0)),
                  pl.BlockSpec((1, d_out), lambda i: (0, 0))],
        out_specs=pl.BlockSpec((block, d_out), lambda i: (i, 0)),
        out_shape=jax.ShapeDtypeStruct((n_nodes, d_out), jnp.float32),
    )(op, dp, b.reshape(1, D), wh, bh.reshape(1, d_out))


@functools.partial(jax.jit, static_argnames=("n_nodes", "n_edges"))
def _sc_edge_layer(xl, xr, eaf, src1, dst1, att_b, *, n_nodes, n_edges):
    ngroups = n_edges // G
    trips = ngroups // NW

    mesh = plsc.VectorSubcoreMesh(core_axis_name="c", subcore_axis_name="s",
                                  num_cores=NC, num_subcores=NS)

    def body(xl_hbm, xr_hbm, ea_hbm, src_hbm, dst_hbm, attb_hbm,
             out_hbm, den_hbm,
             mb, xb0, xb1, rb0, rb1, sb,
             six0, six1, dix0, dix1, dsx0, dsx1,
             ex0, ex1, pb, att_v,
             out_sp, den_sp,
             sem_ea, sem_xl0, sem_xl1, sem_xr0, sem_xr1,
             sem_si0, sem_si1, sem_di0, sem_di1,
             sem_d0, sem_d1, sem_s):
        xbuf = (xb0, xb1)
        rbuf = (rb0, rb1)
        six = (six0, six1)
        dix = (dix0, dix1)
        dsx = (dsx0, dsx1)
        exv = (ex0, ex1)
        sem_xl = (sem_xl0, sem_xl1)
        sem_xr = (sem_xr0, sem_xr1)
        sem_si = (sem_si0, sem_si1)
        sem_di = (sem_di0, sem_di1)
        sem_d = (sem_d0, sem_d1)

        cid = lax.axis_index("c")
        sid = lax.axis_index("s")
        w = sid * NC + cid
        gb = w * trips

        pltpu.sync_copy(attb_hbm, att_v)

        def zrow(e, _):
            for k in range(D // L):
                sb[e, pl.ds(k * L, L)] = jnp.zeros((L,), jnp.float32)
            return 0
        lax.fori_loop(0, G, zrow, 0)

        def zpb(i, _):
            pb[pl.ds(i * L, L)] = jnp.zeros((L,), jnp.float32)
            return 0
        lax.fori_loop(0, G, zpb, 0)

        rps = n_nodes // NS
        for k in range(rps // G):
            pltpu.sync_copy(
                sb,
                out_sp.at[pl.ds(sid * rps + k * G, G)])

        @pl.when(sid == 0)
        def _():
            for k in range(n_nodes // (G * L)):
                pltpu.sync_copy(pb, den_sp.at[pl.ds(k * (G * L), G * L)])

        plsc.subcore_barrier()

        att_regs = tuple(att_v[s] for s in range(8))

        def si_cp(i, b):
            return pltpu.make_async_copy(
                src_hbm.at[pl.ds((gb + i) * G, G)], six[b], sem_si[b])

        def di_cp(i, b):
            return pltpu.make_async_copy(
                dst_hbm.at[pl.ds((gb + i) * G, G)], dix[b], sem_di[b])

        def ea_cp(i):
            return pltpu.make_async_copy(
                ea_hbm.at[pl.ds((gb + i) * G * DP, G * DP)], mb, sem_ea)

        def xl_cp(b):
            return pltpu.make_async_copy(
                xl_hbm.at[six[b]], rbuf[b], sem_xl[b])

        def xr_cp(b):
            return pltpu.make_async_copy(
                xr_hbm.at[dix[b]], xbuf[b], sem_xr[b])

        def out_cp(b):
            return pltpu.make_async_copy(
                sb, out_sp.at[dsx[b]], sem_s)

        def den_cp(b):
            return pltpu.make_async_copy(
                exv[b], den_sp.at[dsx[b]], sem_d[b])

        def group_body(g, b, wait_den, wait_out, do_prefetch, do_ea):
            ea_cp(g).wait()
            xl_cp(b).wait()
            xr_cp(b).wait()

            if wait_den:
                den_cp(b).wait()
            for k in range(G // L):
                dsx[b][pl.ds(k * L, L)] = dix[b][pl.ds(k * L, L)]

            if do_prefetch:
                si_cp(g + 2, b).start()
                di_cp(g + 2, b).start()

            def dot_body(e, att_t):
                acc = jnp.zeros((L,), jnp.float32)
                for k in range(4):
                    mv = plsc.bitcast(mb[pl.ds(e * DP + k * L, L)],
                                      jnp.bfloat16)
                    lo, hi = plsc.unpack(
                        mv, format=plsc.PackFormat.INTERLEAVED)
                    for half, ev, ai in ((0, lo, 2 * k), (1, hi, 2 * k + 1)):
                        off = k * 2 * L + half * L
                        vv = (ev + rbuf[b][e, pl.ds(off, L)]
                              + xbuf[b][e, pl.ds(off, L)])
                        lr = (jnp.maximum(vv, 0.0)
                              + NEG_SLOPE * jnp.minimum(vv, 0.0))
                        acc = acc + lr * att_t[ai]
                pb[pl.ds(e * L, L)] = acc
                return att_t

            lax.fori_loop(0, G, dot_body, att_regs)

            if do_ea:
                ea_cp(g + 1).start()

            lane = lax.iota(jnp.int32, L)
            for s in range(G // L):
                rowbase = (lane + s * L) * L
                tot = plsc.load_gather(pb, [rowbase])
                for t in range(1, L):
                    tot = tot + plsc.load_gather(pb, [rowbase + t])
                exv[b][pl.ds(s * L, L)] = jnp.exp(tot)

            if wait_out:
                out_cp(0).wait()

            def scale_body(e, _):
                bv = plsc.load_gather(exv[b], [jnp.full((L,), e, jnp.int32)])
                for k in range(D // L):
                    sb[e, pl.ds(k * L, L)] = \
                        rbuf[b][e, pl.ds(k * L, L)] * bv
                return 0

            lax.fori_loop(0, G, scale_body, 0)

            if do_prefetch:
                si_cp(g + 2, b).wait()
                di_cp(g + 2, b).wait()
                xl_cp(b).start()
                xr_cp(b).start()

            den_cp(b).start(add=True)
            out_cp(b).start(add=True)

        si_cp(0, 0).start()
        di_cp(0, 0).start()
        si_cp(1, 1).start()
        di_cp(1, 1).start()
        ea_cp(0).start()
        si_cp(0, 0).wait()
        di_cp(0, 0).wait()
        xl_cp(0).start()
        xr_cp(0).start()
        si_cp(1, 1).wait()
        di_cp(1, 1).wait()
        xl_cp(1).start()
        xr_cp(1).start()

        group_body(0, 0, False, False, True, True)
        group_body(1, 1, False, True, True, True)

        def jbody(j, _):
            group_body(2 * j, 0, True, True, True, True)
            group_body(2 * j + 1, 1, True, True, True, True)
            return 0

        lax.fori_loop(1, trips // 2 - 1, jbody, 0)
        group_body(trips - 2, 0, True, True, False, True)
        group_body(trips - 1, 1, True, True, False, False)

        den_cp(0).wait()
        den_cp(1).wait()
        out_cp(0).wait()
        plsc.subcore_barrier()

        @pl.when(sid == 0)
        def _():
            pltpu.sync_copy(out_sp, out_hbm.at[cid])
            pltpu.sync_copy(den_sp, den_hbm.at[cid])

    run = pl.kernel(
        body,
        out_type=(jax.ShapeDtypeStruct((NC, n_nodes, D), jnp.float32),
                  jax.ShapeDtypeStruct((NC, n_nodes), jnp.float32)),
        mesh=mesh,
        compiler_params=pltpu.CompilerParams(needs_layout_passes=False),
        scratch_types=[
            pltpu.VMEM((G * DP,), jnp.float32),
            pltpu.VMEM((G, D), jnp.float32),
            pltpu.VMEM((G, D), jnp.float32),
            pltpu.VMEM((G, D), jnp.float32),
            pltpu.VMEM((G, D), jnp.float32),
            pltpu.VMEM((G, D), jnp.float32),
            pltpu.VMEM((G,), jnp.int32),
            pltpu.VMEM((G,), jnp.int32),
            pltpu.VMEM((G,), jnp.int32),
            pltpu.VMEM((G,), jnp.int32),
            pltpu.VMEM((G,), jnp.int32),
            pltpu.VMEM((G,), jnp.int32),
            pltpu.VMEM((G,), jnp.float32),
            pltpu.VMEM((G,), jnp.float32),
            pltpu.VMEM((G * L,), jnp.float32),
            pltpu.VMEM((D // L, L), jnp.float32),
            pltpu.VMEM_SHARED((n_nodes, D), jnp.float32),
            pltpu.VMEM_SHARED((n_nodes,), jnp.float32),
            pltpu.SemaphoreType.DMA, pltpu.SemaphoreType.DMA,
            pltpu.SemaphoreType.DMA, pltpu.SemaphoreType.DMA,
            pltpu.SemaphoreType.DMA, pltpu.SemaphoreType.DMA,
            pltpu.SemaphoreType.DMA, pltpu.SemaphoreType.DMA,
            pltpu.SemaphoreType.DMA, pltpu.SemaphoreType.DMA,
            pltpu.SemaphoreType.DMA, pltpu.SemaphoreType.DMA,
        ],
    )
    return run(xl, xr, eaf, src1, dst1, att_b)


def kernel(x, edge_index, edge_attr, Wl1, Wr1, We1, att1, b1,
           Wl2, Wr2, We2, att2, b2, Wh, bh):
    n_nodes = x.shape[0]
    n_edges = edge_index.shape[1]
    d_e = edge_attr.shape[1]

    ngroups = -(-n_edges // G)
    tpt = -(-ngroups // NW)
    tpt += tpt % 2
    e_pad = NW * tpt * G
    np_nodes = n_nodes + NPAD
    pe = e_pad - n_edges
    pad_idx = n_nodes + (jnp.arange(pe, dtype=jnp.int32) % NPAD)
    src1 = jnp.concatenate([edge_index[0], pad_idx])
    dst1 = jnp.concatenate([edge_index[1], pad_idx])
    ea_in = jnp.concatenate([edge_attr,
                             jnp.zeros((pe, d_e), jnp.float32)])
    x_pad = jnp.concatenate([x, jnp.zeros((NPAD, D), jnp.float32)])

    att_b1 = att1.reshape(D // L, L)
    att_b2 = att2.reshape(D // L, L)

    j = jnp.arange(L, dtype=jnp.int32)
    chunk = jnp.stack([j, j + L], axis=1).reshape(2 * L)
    perm2 = jnp.concatenate([k * 2 * L + chunk for k in range(D // (2 * L))])

    def pack_flat(a):
        m, n = a.shape
        return lax.bitcast_convert_type(
            a.reshape(m, n // 2, 2), jnp.float32).reshape(m * (n // 2))

    ea1, ea2 = _mm2(ea_in, We1[:, perm2], We2[:, perm2], 2048, jnp.bfloat16)
    xl1, xr1 = _mm2(x_pad, Wl1, Wr1, 512, jnp.float32)
    op1, dp1 = _sc_edge_layer(xl1, xr1, pack_flat(ea1),
                              src1, dst1, att_b1,
                              n_nodes=np_nodes, n_edges=e_pad)
    xl2, xr2 = _combine_mm2(op1, dp1.reshape(NC, np_nodes, 1), b1,
                            Wl2, Wr2, 512)
    op2, dp2 = _sc_edge_layer(xl2, xr2, pack_flat(ea2),
                              src1, dst1, att_b2,
                              n_nodes=np_nodes, n_edges=e_pad)
    out = _final(op2, dp2.reshape(NC, np_nodes, 1), b2, Wh, bh, 512)
    return out[:n_nodes]


# speedup vs baseline: 2.0665x; 2.0665x over previous
import functools

import jax
import jax.numpy as jnp
from jax import lax
from jax.experimental import pallas as pl
from jax.experimental.pallas import tpu as pltpu
from jax.experimental.pallas import tpu_sc as plsc

NEG_SLOPE = 0.2
EPS = 1e-16
NC = 2
NS = 16
NW = NC * NS
L = 16
G = 64
D = 128
NPAD = 240


def _mm2_body(x_ref, wa_ref, wb_ref, oa_ref, ob_ref):
    xv = x_ref[...]
    oa_ref[...] = jnp.dot(xv, wa_ref[...], preferred_element_type=jnp.float32)
    ob_ref[...] = jnp.dot(xv, wb_ref[...], preferred_element_type=jnp.float32)


def _mm2(x, wa, wb, block):
    m, k = x.shape
    n = wa.shape[1]
    return pl.pallas_call(
        _mm2_body,
        grid=(m // block,),
        in_specs=[pl.BlockSpec((block, k), lambda i: (i, 0)),
                  pl.BlockSpec((k, n), lambda i: (0, 0)),
                  pl.BlockSpec((k, n), lambda i: (0, 0))],
        out_specs=[pl.BlockSpec((block, n), lambda i: (i, 0)),
                   pl.BlockSpec((block, n), lambda i: (i, 0))],
        out_shape=[jax.ShapeDtypeStruct((m, n), jnp.float32),
                   jax.ShapeDtypeStruct((m, n), jnp.float32)],
    )(x, wa, wb)


def _combine_body(op_ref, dp_ref, b_ref, wa_ref, wb_ref, oa_ref, ob_ref):
    o = op_ref[0] + op_ref[1]
    den = dp_ref[0] + dp_ref[1] + EPS
    h = jax.nn.relu(o / den + b_ref[...])
    oa_ref[...] = jnp.dot(h, wa_ref[...], preferred_element_type=jnp.float32)
    ob_ref[...] = jnp.dot(h, wb_ref[...], preferred_element_type=jnp.float32)


def _combine_mm2(op, dp, b, wa, wb, block):
    n_nodes = op.shape[1]
    return pl.pallas_call(
        _combine_body,
        grid=(n_nodes // block,),
        in_specs=[pl.BlockSpec((2, block, D), lambda i: (0, i, 0)),
                  pl.BlockSpec((2, block, 1), lambda i: (0, i, 0)),
                  pl.BlockSpec((1, D), lambda i: (0, 0)),
                  pl.BlockSpec((D, D), lambda i: (0, 0)),
                  pl.BlockSpec((D, D), lambda i: (0, 0))],
        out_specs=[pl.BlockSpec((block, D), lambda i: (i, 0)),
                   pl.BlockSpec((block, D), lambda i: (i, 0))],
        out_shape=[jax.ShapeDtypeStruct((n_nodes, D), jnp.float32),
                   jax.ShapeDtypeStruct((n_nodes, D), jnp.float32)],
    )(op, dp, b.reshape(1, D), wa, wb)


def _final_body(op_ref, dp_ref, b_ref, wh_ref, bh_ref, o_ref):
    o = op_ref[0] + op_ref[1]
    den = dp_ref[0] + dp_ref[1] + EPS
    h = jax.nn.relu(o / den + b_ref[...])
    o_ref[...] = (jnp.dot(h, wh_ref[...], preferred_element_type=jnp.float32)
                  + bh_ref[...])


def _final(op, dp, b, wh, bh, block):
    n_nodes = op.shape[1]
    d_out = wh.shape[1]
    return pl.pallas_call(
        _final_body,
        grid=(n_nodes // block,),
        in_specs=[pl.BlockSpec((2, block, D), lambda i: (0, i, 0)),
                  pl.BlockSpec((2, block, 1), lambda i: (0, i, 0)),
                  pl.BlockSpec((1, D), lambda i: (0, 0)),
                  pl.BlockSpec((D, d_out), lambda i: (0, 0)),
                  pl.BlockSpec((1, d_out), lambda i: (0, 0))],
        out_specs=pl.BlockSpec((block, d_out), lambda i: (i, 0)),
        out_shape=jax.ShapeDtypeStruct((n_nodes, d_out), jnp.float32),
    )(op, dp, b.reshape(1, D), wh, bh.reshape(1, d_out))


@functools.partial(jax.jit, static_argnames=("n_nodes", "n_edges"))
def _sc_edge_layer(xl, xr, ea, src1, dst1, att_b, *, n_nodes, n_edges):
    ngroups = n_edges // G
    trips = ngroups // NW

    mesh = plsc.VectorSubcoreMesh(core_axis_name="c", subcore_axis_name="s",
                                  num_cores=NC, num_subcores=NS)

    def body(xl_hbm, xr_hbm, ea_hbm, src_hbm, dst_hbm, attb_hbm,
             out_hbm, den_hbm,
             mb0, mb1, rb0, rb1, sb,
             six0, six1, dix0, dix1, dsx0, dsx1,
             ex0, ex1, pb, att_v,
             out_sp, den_sp,
             sem_ea0, sem_ea1, sem_xl0, sem_xl1, sem_xr0, sem_xr1,
             sem_si0, sem_si1, sem_di0, sem_di1,
             sem_d0, sem_d1, sem_s):
        mbuf = (mb0, mb1)
        rbuf = (rb0, rb1)
        six = (six0, six1)
        dix = (dix0, dix1)
        dsx = (dsx0, dsx1)
        exv = (ex0, ex1)
        sem_ea = (sem_ea0, sem_ea1)
        sem_xl = (sem_xl0, sem_xl1)
        sem_xr = (sem_xr0, sem_xr1)
        sem_si = (sem_si0, sem_si1)
        sem_di = (sem_di0, sem_di1)
        sem_d = (sem_d0, sem_d1)

        cid = lax.axis_index("c")
        sid = lax.axis_index("s")
        w = sid * NC + cid
        gb = w * trips

        pltpu.sync_copy(attb_hbm, att_v)

        def zrow(e, _):
            for k in range(D // L):
                mb0[e, pl.ds(k * L, L)] = jnp.zeros((L,), jnp.float32)
            return 0
        lax.fori_loop(0, G, zrow, 0)

        def zpb(i, _):
            pb[pl.ds(i * L, L)] = jnp.zeros((L,), jnp.float32)
            return 0
        lax.fori_loop(0, (G * L) // L, zpb, 0)

        rps = n_nodes // NS
        for k in range(rps // G):
            pltpu.sync_copy(
                mb0,
                out_sp.at[pl.ds(sid * rps + k * G, G)])

        @pl.when(sid == 0)
        def _():
            for k in range(n_nodes // (G * L)):
                pltpu.sync_copy(pb, den_sp.at[pl.ds(k * (G * L), G * L)])

        plsc.subcore_barrier()

        att_regs = tuple(att_v[s] for s in range(8))

        def si_cp(i, b):
            return pltpu.make_async_copy(
                src_hbm.at[pl.ds((gb + i) * G, G)], six[b], sem_si[b])

        def di_cp(i, b):
            return pltpu.make_async_copy(
                dst_hbm.at[pl.ds((gb + i) * G, G)], dix[b], sem_di[b])

        def ea_cp(i, b):
            return pltpu.make_async_copy(
                ea_hbm.at[pl.ds((gb + i) * G, G)], mbuf[b], sem_ea[b])

        def xl_cp(b):
            return pltpu.make_async_copy(
                xl_hbm.at[six[b]], rbuf[b], sem_xl[b])

        def xr_cp(b):
            return pltpu.make_async_copy(
                xr_hbm.at[dix[b]], mbuf[b], sem_xr[b])

        def out_cp(b):
            return pltpu.make_async_copy(
                sb, out_sp.at[dsx[b]], sem_s)

        def den_cp(b):
            return pltpu.make_async_copy(
                exv[b], den_sp.at[dsx[b]], sem_d[b])

        def group_body(g, b, wait_den, wait_out, do_prefetch):
            xl_cp(b).wait()
            xr_cp(b).wait()

            if wait_den:
                den_cp(b).wait()
            for k in range(G // L):
                dsx[b][pl.ds(k * L, L)] = dix[b][pl.ds(k * L, L)]

            if do_prefetch:
                si_cp(g + 2, b).start()
                di_cp(g + 2, b).start()

            def dot_body(e, att_t):
                acc = jnp.zeros((L,), jnp.float32)
                for k in range(8):
                    v = mbuf[b][e, pl.ds(k * L, L)] \
                        + rbuf[b][e, pl.ds(k * L, L)]
                    lr = (jnp.maximum(v, 0.0)
                          + NEG_SLOPE * jnp.minimum(v, 0.0))
                    acc = acc + lr * att_t[k]
                pb[pl.ds(e * L, L)] = acc
                return att_t

            lax.fori_loop(0, G, dot_body, att_regs)

            if do_prefetch:
                ea_cp(g + 2, b).start()

            lane = lax.iota(jnp.int32, L)
            for s in range(G // L):
                rowbase = (lane + s * L) * L
                tot = plsc.load_gather(pb, [rowbase])
                for t in range(1, L):
                    tot = tot + plsc.load_gather(pb, [rowbase + t])
                exv[b][pl.ds(s * L, L)] = jnp.exp(tot)

            if wait_out:
                out_cp(0).wait()

            def scale_body(e, _):
                bv = plsc.load_gather(exv[b], [jnp.full((L,), e, jnp.int32)])
                for k in range(8):
                    sb[e, pl.ds(k * L, L)] = \
                        rbuf[b][e, pl.ds(k * L, L)] * bv
                return 0

            lax.fori_loop(0, G, scale_body, 0)

            if do_prefetch:
                si_cp(g + 2, b).wait()
                di_cp(g + 2, b).wait()
                xl_cp(b).start()
                ea_cp(g + 2, b).wait()
                xr_cp(b).start(add=True)

            den_cp(b).start(add=True)
            out_cp(b).start(add=True)

        si_cp(0, 0).start()
        di_cp(0, 0).start()
        si_cp(1, 1).start()
        di_cp(1, 1).start()
        ea_cp(0, 0).start()
        ea_cp(1, 1).start()
        si_cp(0, 0).wait()
        di_cp(0, 0).wait()
        xl_cp(0).start()
        ea_cp(0, 0).wait()
        xr_cp(0).start(add=True)
        si_cp(1, 1).wait()
        di_cp(1, 1).wait()
        xl_cp(1).start()
        ea_cp(1, 1).wait()
        xr_cp(1).start(add=True)

        group_body(0, 0, False, False, True)
        group_body(1, 1, False, True, True)

        def jbody(j, _):
            group_body(2 * j, 0, True, True, True)
            group_body(2 * j + 1, 1, True, True, True)
            return 0

        lax.fori_loop(1, trips // 2 - 1, jbody, 0)
        group_body(trips - 2, 0, True, True, False)
        group_body(trips - 1, 1, True, True, False)

        den_cp(0).wait()
        den_cp(1).wait()
        out_cp(0).wait()
        plsc.subcore_barrier()

        @pl.when(sid == 0)
        def _():
            pltpu.sync_copy(out_sp, out_hbm.at[cid])
            pltpu.sync_copy(den_sp, den_hbm.at[cid])

    run = pl.kernel(
        body,
        out_type=(jax.ShapeDtypeStruct((NC, n_nodes, D), jnp.float32),
                  jax.ShapeDtypeStruct((NC, n_nodes), jnp.float32)),
        mesh=mesh,
        compiler_params=pltpu.CompilerParams(needs_layout_passes=False),
        scratch_types=[
            pltpu.VMEM((G, D), jnp.float32),
            pltpu.VMEM((G, D), jnp.float32),
            pltpu.VMEM((G, D), jnp.float32),
            pltpu.VMEM((G, D), jnp.float32),
            pltpu.VMEM((G, D), jnp.float32),
            pltpu.VMEM((G,), jnp.int32),
            pltpu.VMEM((G,), jnp.int32),
            pltpu.VMEM((G,), jnp.int32),
            pltpu.VMEM((G,), jnp.int32),
            pltpu.VMEM((G,), jnp.int32),
            pltpu.VMEM((G,), jnp.int32),
            pltpu.VMEM((G,), jnp.float32),
            pltpu.VMEM((G,), jnp.float32),
            pltpu.VMEM((G * L,), jnp.float32),
            pltpu.VMEM((D // L, L), jnp.float32),
            pltpu.VMEM_SHARED((n_nodes, D), jnp.float32),
            pltpu.VMEM_SHARED((n_nodes,), jnp.float32),
            pltpu.SemaphoreType.DMA, pltpu.SemaphoreType.DMA,
            pltpu.SemaphoreType.DMA, pltpu.SemaphoreType.DMA,
            pltpu.SemaphoreType.DMA, pltpu.SemaphoreType.DMA,
            pltpu.SemaphoreType.DMA, pltpu.SemaphoreType.DMA,
            pltpu.SemaphoreType.DMA, pltpu.SemaphoreType.DMA,
            pltpu.SemaphoreType.DMA, pltpu.SemaphoreType.DMA,
            pltpu.SemaphoreType.DMA,
        ],
    )
    return run(xl, xr, ea, src1, dst1, att_b)


def kernel(x, edge_index, edge_attr, Wl1, Wr1, We1, att1, b1,
           Wl2, Wr2, We2, att2, b2, Wh, bh):
    n_nodes = x.shape[0]
    n_edges = edge_index.shape[1]
    d_e = edge_attr.shape[1]

    ngroups = -(-n_edges // G)
    tpt = -(-ngroups // NW)
    tpt += tpt % 2
    e_pad = NW * tpt * G
    np_nodes = n_nodes + NPAD
    pe = e_pad - n_edges
    pad_idx = n_nodes + (jnp.arange(pe, dtype=jnp.int32) % NPAD)
    src1 = jnp.concatenate([edge_index[0], pad_idx])
    dst1 = jnp.concatenate([edge_index[1], pad_idx])
    ea_in = jnp.concatenate([edge_attr,
                             jnp.zeros((pe, d_e), jnp.float32)])
    x_pad = jnp.concatenate([x, jnp.zeros((NPAD, D), jnp.float32)])

    att_b1 = att1.reshape(D // L, L)
    att_b2 = att2.reshape(D // L, L)

    ea1, ea2 = _mm2(ea_in, We1, We2, 2048)
    xl1, xr1 = _mm2(x_pad, Wl1, Wr1, 512)
    op1, dp1 = _sc_edge_layer(xl1, xr1, ea1, src1, dst1, att_b1,
                              n_nodes=np_nodes, n_edges=e_pad)
    xl2, xr2 = _combine_mm2(op1, dp1.reshape(NC, np_nodes, 1), b1,
                            Wl2, Wr2, 512)
    op2, dp2 = _sc_edge_layer(xl2, xr2, ea2, src1, dst1, att_b2,
                              n_nodes=np_nodes, n_edges=e_pad)
    out = _final(op2, dp2.reshape(NC, np_nodes, 1), b2, Wh, bh, 512)
    return out[:n_nodes]
